# baseline (speedup 1.0000x reference)
import functools

import jax
import jax.numpy as jnp
from jax import lax
from jax.experimental import pallas as pl
from jax.experimental.pallas import tpu as pltpu
from jax.experimental.pallas import tpu_sc as plsc

E = 8
K = 2
T = 2048
H = 768
I = 2048
BLK = 256
NB = (T * K) // BLK + E
NPAD = NB * BLK
NW = 32
RPW = T // NW
NEG = -1e30


def _router_body(flat_ref, gate_ref, pos_ref, w0r_ref, w1r_ref,
                 bexp_ref, lb_ref):
    flat = flat_ref[...]
    gate = gate_ref[...]
    scores = lax.dot_general(flat, gate, (((1,), (1,)), ((), ())),
                             preferred_element_type=jnp.float32)
    eidx = lax.broadcasted_iota(jnp.int32, (T, E), 1)
    m1 = jnp.max(scores, axis=1, keepdims=True)
    a1 = jnp.min(jnp.where(scores == m1, eidx, E), axis=1, keepdims=True)
    masked = jnp.where(eidx == a1, NEG, scores)
    m2 = jnp.max(masked, axis=1, keepdims=True)
    a2 = jnp.min(jnp.where(masked == m2, eidx, E), axis=1, keepdims=True)
    r = jnp.exp(m2 - m1)
    w0c = 1.0 / (1.0 + r)
    w1c = r / (1.0 + r)
    ones128 = jnp.ones((1, 128), jnp.float32)
    w0r_ref[...] = w0c * ones128
    w1r_ref[...] = w1c * ones128
    ex = jnp.exp(scores - m1)
    gates = ex / jnp.sum(ex, axis=1, keepdims=True)
    onehot1 = (eidx == a1).astype(jnp.float32)
    onehot2 = (eidx == a2).astype(jnp.float32)
    row_cnt = onehot1 + onehot2
    counts = jnp.sum(row_cnt, axis=0, keepdims=True)
    router_prob = jnp.mean(gates, axis=0, keepdims=True)
    lb = E * jnp.sum((counts / T) * router_prob)
    lb_ref[...] = lb.reshape(1, 1)
    inc = row_cnt
    sh = 1
    while sh < T:
        inc = inc + jnp.pad(inc, ((sh, 0), (0, 0)))[:T]
        sh *= 2
    csum = inc - row_cnt
    padded = jnp.ceil(counts / BLK) * BLK
    pstart = jnp.zeros_like(padded)
    for s in range(1, E):
        pstart = pstart + jnp.pad(padded, ((0, 0), (s, 0)))[:, :E]
    dest = pstart + csum
    pos0 = jnp.sum(onehot1 * dest, axis=1, keepdims=True)
    pos1 = jnp.sum(onehot2 * dest, axis=1, keepdims=True)
    pos_ref[...] = jnp.concatenate([pos0, pos1], axis=1).astype(jnp.int32)
    bstart = lax.broadcasted_iota(jnp.int32, (1, 128), 1).astype(jnp.float32) * BLK
    bexp = jnp.zeros((1, 128), jnp.float32)
    for e in range(1, E):
        pe = lax.slice(pstart, (0, e), (1, e + 1))
        bexp = bexp + (pe <= bstart).astype(jnp.float32)
    last_present = jnp.zeros((1, 1), jnp.float32)
    for e in range(E):
        ge = lax.slice(counts, (0, e), (1, e + 1))
        last_present = jnp.where(ge > 0, float(e), last_present)
    bexp = jnp.minimum(bexp, last_present)
    lane = lax.broadcasted_iota(jnp.int32, (1, 128), 1)
    bexp_sh = jnp.pad(bexp, ((0, 0), (1, 0)))[:, :128]
    trans = jnp.where((lane == 0) | (bexp != bexp_sh), 1.0, 0.0)
    run_id = trans
    for s in (1, 2, 4, 8, 16, 32, 64):
        run_id = run_id + jnp.pad(run_id, ((0, 0), (s, 0)))[:, :128]
    par = run_id - 2.0 * jnp.floor(run_id / 2.0)
    nxt = jnp.full((1, 128), -1.0)
    for e in range(E - 1, -1, -1):
        ge = lax.slice(counts, (0, e), (1, e + 1))
        nxt = jnp.where((bexp < float(e)) & (ge > 0), float(e), nxt)
    total_used = jnp.sum(padded).reshape(1, 1)
    active = (bstart < total_used).astype(jnp.float32)
    xidx = jnp.where(active > 0, lane.astype(jnp.float32), 0.0)
    zero = jnp.zeros((1, 128), jnp.float32)
    sched = jnp.concatenate(
        [bexp, trans, par, nxt, active, xidx, zero, zero], axis=0)
    bexp_ref[...] = sched.astype(jnp.int32)


def _mlp_body(s_ref, x_ref, ws_ref, w1_ref, w2_ref, w3_ref, o_ref,
              w1b, w2b, w3b, sems):
    i = pl.program_id(0)
    e = s_ref[0, i]
    trans = s_ref[1, i]
    par = s_ref[2, i]
    nxt = s_ref[3, i]
    act = s_ref[4, i]

    def _fetch(dst_e, buf):
        pltpu.make_async_copy(w1_ref.at[dst_e], w1b.at[buf], sems.at[buf, 0]).start()
        pltpu.make_async_copy(w2_ref.at[dst_e], w2b.at[buf], sems.at[buf, 1]).start()
        pltpu.make_async_copy(w3_ref.at[dst_e], w3b.at[buf], sems.at[buf, 2]).start()

    @pl.when(i == 0)
    def _():
        _fetch(e, par)

    @pl.when((trans == 1) & (nxt >= 0))
    def _():
        _fetch(nxt, 1 - par)

    def _compute():
        x = x_ref[...]
        h1 = lax.dot_general(x, w1b[par],
                             (((1,), (1,)), ((), ())),
                             preferred_element_type=jnp.float32)
        h2 = lax.dot_general(x, w2b[par],
                             (((1,), (1,)), ((), ())),
                             preferred_element_type=jnp.float32)
        g = (h1 * jax.nn.sigmoid(h1)) * h2
        y = lax.dot_general(g, w3b[par],
                            (((1,), (1,)), ((), ())),
                            preferred_element_type=jnp.float32)
        o_ref[...] = y * ws_ref[:, 0:1]

    @pl.when(trans == 1)
    def _():
        x = x_ref[...]
        pltpu.make_async_copy(w1_ref.at[e], w1b.at[par], sems.at[par, 0]).wait()
        h1 = lax.dot_general(x, w1b[par],
                             (((1,), (1,)), ((), ())),
                             preferred_element_type=jnp.float32)
        pltpu.make_async_copy(w2_ref.at[e], w2b.at[par], sems.at[par, 1]).wait()
        h2 = lax.dot_general(x, w2b[par],
                             (((1,), (1,)), ((), ())),
                             preferred_element_type=jnp.float32)
        g = (h1 * jax.nn.sigmoid(h1)) * h2
        pltpu.make_async_copy(w3_ref.at[e], w3b.at[par], sems.at[par, 2]).wait()
        y = lax.dot_general(g, w3b[par],
                            (((1,), (1,)), ((), ())),
                            preferred_element_type=jnp.float32)
        o_ref[...] = y * ws_ref[:, 0:1]

    @pl.when((trans == 0) & (act == 1))
    def _():
        _compute()


@functools.cache
def _sc_dispatch():
    mesh = plsc.VectorSubcoreMesh(core_axis_name="c", subcore_axis_name="s")

    @functools.partial(
        pl.kernel,
        out_type=(jax.ShapeDtypeStruct((NPAD, H), jnp.float32),
                  jax.ShapeDtypeStruct((NPAD, 128), jnp.float32)),
        mesh=mesh,
        scratch_types=[
            pltpu.VMEM((RPW,), jnp.int32),
            pltpu.VMEM((RPW,), jnp.int32),
            pltpu.VMEM((RPW, H), jnp.float32),
            pltpu.VMEM((RPW, 128), jnp.float32),
            pltpu.VMEM((RPW, 128), jnp.float32),
            pltpu.SemaphoreType.DMA,
            pltpu.SemaphoreType.DMA,
            pltpu.SemaphoreType.DMA,
            pltpu.SemaphoreType.DMA,
        ],
    )
    def body(flat_hbm, pos0_hbm, pos1_hbm, w0r_hbm, w1r_hbm, xs_hbm, ws_hbm,
             idx0_v, idx1_v, rows_v, wrep0_v, wrep1_v,
             sem0, sem1, sem2, sem3):
        wid = lax.axis_index("s") * 2 + lax.axis_index("c")
        base = wid * RPW
        pltpu.sync_copy(pos0_hbm.at[pl.ds(base, RPW)], idx0_v)
        pltpu.sync_copy(pos1_hbm.at[pl.ds(base, RPW)], idx1_v)
        pltpu.sync_copy(w0r_hbm.at[pl.ds(base, RPW)], wrep0_v)
        pltpu.sync_copy(w1r_hbm.at[pl.ds(base, RPW)], wrep1_v)
        pltpu.sync_copy(flat_hbm.at[pl.ds(base, RPW)], rows_v)
        c0 = pltpu.async_copy(rows_v, xs_hbm.at[idx0_v], sem0)
        c1 = pltpu.async_copy(rows_v, xs_hbm.at[idx1_v], sem1)
        c2 = pltpu.async_copy(wrep0_v, ws_hbm.at[idx0_v], sem2)
        c3 = pltpu.async_copy(wrep1_v, ws_hbm.at[idx1_v], sem3)
        c0.wait()
        c1.wait()
        c2.wait()
        c3.wait()

    return body


@functools.cache
def _sc_gather():
    mesh = plsc.VectorSubcoreMesh(core_axis_name="c", subcore_axis_name="s")

    @functools.partial(
        pl.kernel,
        out_type=jax.ShapeDtypeStruct((T, H), jnp.float32),
        mesh=mesh,
        scratch_types=[
            pltpu.VMEM((RPW,), jnp.int32),
            pltpu.VMEM((RPW,), jnp.int32),
            pltpu.VMEM((RPW, H), jnp.float32),
            pltpu.VMEM((RPW, H), jnp.float32),
            pltpu.SemaphoreType.DMA,
            pltpu.SemaphoreType.DMA,
        ],
    )
    def body(ys_hbm, pos0_hbm, pos1_hbm, out_hbm,
             idx0_v, idx1_v, rows0_v, rows1_v, sem0, sem1):
        wid = lax.axis_index("s") * 2 + lax.axis_index("c")
        base = wid * RPW
        pltpu.sync_copy(pos0_hbm.at[pl.ds(base, RPW)], idx0_v)
        pltpu.sync_copy(pos1_hbm.at[pl.ds(base, RPW)], idx1_v)
        c0 = pltpu.async_copy(ys_hbm.at[idx0_v], rows0_v, sem0)
        c1 = pltpu.async_copy(ys_hbm.at[idx1_v], rows1_v, sem1)
        c0.wait()
        c1.wait()

        def add_row(j, _):
            for c in range(H // 16):
                sl = pl.ds(c * 16, 16)
                rows0_v[j, sl] = rows0_v[j, sl] + rows1_v[j, sl]
            return 0

        lax.fori_loop(0, RPW, add_row, 0)
        pltpu.sync_copy(rows0_v, out_hbm.at[pl.ds(base, RPW)])

    return body


def kernel(hidden_states, gate_w, w1, w2, w3):
    b, s, h = hidden_states.shape
    flat = hidden_states.reshape(s, h)

    pos, w0r, w1r, sched, lb = pl.pallas_call(
        _router_body,
        out_shape=[
            jax.ShapeDtypeStruct((T, K), jnp.int32),
            jax.ShapeDtypeStruct((T, 128), jnp.float32),
            jax.ShapeDtypeStruct((T, 128), jnp.float32),
            jax.ShapeDtypeStruct((8, 128), jnp.int32),
            jax.ShapeDtypeStruct((1, 1), jnp.float32),
        ],
    )(flat, gate_w)

    pos0 = pos[:, 0]
    pos1 = pos[:, 1]

    xs, ws = _sc_dispatch()(flat, pos0, pos1, w0r, w1r)

    ys = pl.pallas_call(
        _mlp_body,
        grid_spec=pltpu.PrefetchScalarGridSpec(
            num_scalar_prefetch=1,
            grid=(NB,),
            in_specs=[
                pl.BlockSpec((BLK, H), lambda i, s: (s[5, i], 0)),
                pl.BlockSpec((BLK, 128), lambda i, s: (s[5, i], 0)),
                pl.BlockSpec(memory_space=pl.ANY),
                pl.BlockSpec(memory_space=pl.ANY),
                pl.BlockSpec(memory_space=pl.ANY),
            ],
            out_specs=pl.BlockSpec((BLK, H), lambda i, s: (i, 0)),
            scratch_shapes=[
                pltpu.VMEM((2, I, H), jnp.float32),
                pltpu.VMEM((2, I, H), jnp.float32),
                pltpu.VMEM((2, H, I), jnp.float32),
                pltpu.SemaphoreType.DMA((2, 3)),
            ],
        ),
        out_shape=jax.ShapeDtypeStruct((NPAD, H), jnp.float32),
    )(sched, xs, ws, w1, w2, w3)

    out = _sc_gather()(ys, pos0, pos1)

    return out.reshape(b, s, h), lb[0, 0]
